# Initial kernel scaffold; baseline (speedup 1.0000x reference)
import jax
import jax.numpy as jnp
from jax.experimental import pallas as pl


def kernel(features, edge_index, W0, b0, W1, b1, W2, b2):
    raise NotImplementedError("write your pallas kernel here")


# speedup vs baseline: 6.0792x; 6.0792x over previous
import functools

import jax
import jax.numpy as jnp
from jax import lax
from jax.experimental import pallas as pl
from jax.experimental.pallas import tpu as pltpu
from jax.experimental.pallas import tpu_sc as plsc

N = 10000
E = 320000
D_IN = 128
D_H = 128
N_CLASSES = 40

NC = 2
NS = 16
NW = NC * NS
EPW = E // NW
CHUNK = 80
NCHUNK = EPW // CHUNK
N_PAD = 10240
RPW = N_PAD // NS

D0 = 144
D1 = 128
D2 = 48

ROW_BLK = 1000
GRID = N // ROW_BLK


def _make_sc_agg(d):
  mesh = plsc.VectorSubcoreMesh(core_axis_name="c", subcore_axis_name="s",
                                num_cores=NC, num_subcores=NS)

  @functools.partial(
      pl.kernel,
      out_type=jax.ShapeDtypeStruct((NC, N_PAD, d), jnp.float32),
      mesh=mesh,
      scratch_types=[
          pltpu.VMEM((EPW,), jnp.int32),
          pltpu.VMEM((CHUNK,), jnp.int32),
          pltpu.VMEM((CHUNK, d), jnp.float32),
          pltpu.VMEM_SHARED((N_PAD, d), jnp.float32),
          pltpu.SemaphoreType.DMA,
      ],
      compiler_params=pltpu.CompilerParams(use_tc_tiling_on_sc=False),
  )
  def agg(table, src, dst, zeros, out, src_v, dst_v, rows_v, acc, sem):
    cid = lax.axis_index("c")
    sid = lax.axis_index("s")
    wid = sid * NC + cid

    pltpu.sync_copy(zeros, acc.at[pl.ds(sid * RPW, RPW)])
    pltpu.sync_copy(src.at[pl.ds(wid * EPW, EPW)], src_v)
    plsc.subcore_barrier()

    def body(ci, carry):
      pltpu.sync_copy(dst.at[pl.ds(wid * EPW + ci * CHUNK, CHUNK)], dst_v)
      pltpu.async_copy(table.at[src_v.at[pl.ds(ci * CHUNK, CHUNK)]],
                       rows_v, sem).wait()
      pltpu.sync_copy(rows_v, acc.at[dst_v], add=True)
      return carry

    lax.fori_loop(0, NCHUNK, body, 0)
    plsc.subcore_barrier()

    pltpu.sync_copy(acc.at[pl.ds(sid * RPW, RPW)],
                    out.at[cid, pl.ds(sid * RPW, RPW)])

  return agg


_sc_agg_cache = {}


def _agg(table, src, dst, zeros, d):
  if d not in _sc_agg_cache:
    _sc_agg_cache[d] = _make_sc_agg(d)
  return _sc_agg_cache[d](table, src, dst, zeros)


def _mm_body(x_ref, w_ref, o_ref):
  o_ref[...] = jnp.dot(x_ref[...], w_ref[...],
                       preferred_element_type=jnp.float32)


def _tc_matmul(x, w, d_out):
  n, d_in = x.shape
  return pl.pallas_call(
      _mm_body,
      grid=(n // ROW_BLK,),
      in_specs=[
          pl.BlockSpec((ROW_BLK, d_in), lambda i: (i, 0)),
          pl.BlockSpec((d_in, d_out), lambda i: (0, 0)),
      ],
      out_specs=pl.BlockSpec((ROW_BLK, d_out), lambda i: (i, 0)),
      out_shape=jax.ShapeDtypeStruct((n, d_out), jnp.float32),
  )(x, w)


def _stage2_body(p0_ref, p1_ref, b_ref, w_ref, t_ref, dinv_ref):
  s = p0_ref[...] + p1_ref[...]
  deg = s[:, D_H:D_H + 1]
  dinv = 1.0 / jnp.maximum(deg, 1.0)
  h = jnp.maximum(s[:, :D_H] * dinv + b_ref[...], 0.0)
  t_ref[...] = jnp.dot(h, w_ref[...], preferred_element_type=jnp.float32)
  dinv_ref[...] = dinv


def _tc_stage2(p0, p1, b0, w1):
  return pl.pallas_call(
      _stage2_body,
      grid=(GRID,),
      in_specs=[
          pl.BlockSpec((ROW_BLK, D0), lambda i: (i, 0)),
          pl.BlockSpec((ROW_BLK, D0), lambda i: (i, 0)),
          pl.BlockSpec((1, D_H), lambda i: (0, 0)),
          pl.BlockSpec((D_H, D_H), lambda i: (0, 0)),
      ],
      out_specs=[
          pl.BlockSpec((ROW_BLK, D_H), lambda i: (i, 0)),
          pl.BlockSpec((ROW_BLK, 1), lambda i: (i, 0)),
      ],
      out_shape=[
          jax.ShapeDtypeStruct((N, D_H), jnp.float32),
          jax.ShapeDtypeStruct((N, 1), jnp.float32),
      ],
  )(p0, p1, b0, w1)


def _stage3_body(p0_ref, p1_ref, dinv_ref, b_ref, w_ref, t_ref):
  s = p0_ref[...] + p1_ref[...]
  h = jnp.maximum(s * dinv_ref[...] + b_ref[...], 0.0)
  t_ref[...] = jnp.dot(h, w_ref[...], preferred_element_type=jnp.float32)


def _tc_stage3(p0, p1, dinv, b1, w2):
  return pl.pallas_call(
      _stage3_body,
      grid=(GRID,),
      in_specs=[
          pl.BlockSpec((ROW_BLK, D_H), lambda i: (i, 0)),
          pl.BlockSpec((ROW_BLK, D_H), lambda i: (i, 0)),
          pl.BlockSpec((ROW_BLK, 1), lambda i: (i, 0)),
          pl.BlockSpec((1, D_H), lambda i: (0, 0)),
          pl.BlockSpec((D_H, D2), lambda i: (0, 0)),
      ],
      out_specs=pl.BlockSpec((ROW_BLK, D2), lambda i: (i, 0)),
      out_shape=jax.ShapeDtypeStruct((N, D2), jnp.float32),
  )(p0, p1, dinv, b1, w2)


def _stage4_body(p0_ref, p1_ref, dinv_ref, b_ref, o_ref):
  s = p0_ref[...] + p1_ref[...]
  o_ref[...] = s * dinv_ref[...] + b_ref[...]


def _tc_stage4(p0, p1, dinv, b2):
  return pl.pallas_call(
      _stage4_body,
      grid=(GRID,),
      in_specs=[
          pl.BlockSpec((ROW_BLK, D2), lambda i: (i, 0)),
          pl.BlockSpec((ROW_BLK, D2), lambda i: (i, 0)),
          pl.BlockSpec((ROW_BLK, 1), lambda i: (i, 0)),
          pl.BlockSpec((1, D2), lambda i: (0, 0)),
      ],
      out_specs=pl.BlockSpec((ROW_BLK, D2), lambda i: (i, 0)),
      out_shape=jax.ShapeDtypeStruct((N, D2), jnp.float32),
  )(p0, p1, dinv, b2)


def kernel(features, edge_index, W0, b0, W1, b1, W2, b2):
  src = edge_index[0].astype(jnp.int32)
  dst = edge_index[1].astype(jnp.int32)

  xaug = jnp.concatenate(
      [features,
       jnp.ones((N, 1), jnp.float32),
       jnp.zeros((N, 7), jnp.float32)], axis=1)
  w0aug = jnp.zeros((D_IN + 8, D0), jnp.float32)
  w0aug = w0aug.at[:D_IN, :D_H].set(W0)
  w0aug = w0aug.at[D_IN, D_H].set(1.0)

  w2pad = jnp.zeros((D_H, D2), jnp.float32).at[:, :N_CLASSES].set(W2)
  b2pad = jnp.zeros((1, D2), jnp.float32).at[0, :N_CLASSES].set(b2)

  z0 = jnp.zeros((RPW, D0), jnp.float32)
  z1 = jnp.zeros((RPW, D1), jnp.float32)
  z2 = jnp.zeros((RPW, D2), jnp.float32)

  t0 = _tc_matmul(xaug, w0aug, D0)
  p = _agg(t0, src, dst, z0, D0)
  t1, dinv = _tc_stage2(p[0, :N], p[1, :N], b0.reshape(1, D_H), W1)
  p = _agg(t1, src, dst, z1, D1)
  t2 = _tc_stage3(p[0, :N], p[1, :N], dinv, b1.reshape(1, D_H), w2pad)
  p = _agg(t2, src, dst, z2, D2)
  out = _tc_stage4(p[0, :N], p[1, :N], dinv, b2pad)
  return out[:, :N_CLASSES]


# speedup vs baseline: 10.7506x; 1.7684x over previous
import functools

import jax
import jax.numpy as jnp
from jax import lax
from jax.experimental import pallas as pl
from jax.experimental.pallas import tpu as pltpu
from jax.experimental.pallas import tpu_sc as plsc

N = 10000
E = 320000
D_IN = 128
D_H = 128
N_CLASSES = 40

NC = 2
NS = 16
NW = NC * NS
EPW = E // NW
RPW = N // NS

D0 = 144
D1 = 128
D2 = 48

ROW_BLK = 1000
GRID = N // ROW_BLK


NBUF = 5
ILEAD = 3
SLAG = 2
_CHUNK_BY_D = {144: 40, 128: 40, 48: 80}


def _make_sc_agg(d):
  chunk = _CHUNK_BY_D[d]
  nchunk = EPW // chunk
  mesh = plsc.VectorSubcoreMesh(core_axis_name="c", subcore_axis_name="s",
                                num_cores=NC, num_subcores=NS)

  @functools.partial(
      pl.kernel,
      out_type=jax.ShapeDtypeStruct((NC, N, d), jnp.float32),
      mesh=mesh,
      scratch_types=[
          [pltpu.VMEM((chunk,), jnp.int32) for _ in range(NBUF)],
          [pltpu.VMEM((chunk,), jnp.int32) for _ in range(NBUF)],
          [pltpu.VMEM((chunk, d), jnp.float32) for _ in range(NBUF)],
          pltpu.SemaphoreType.DMA((NBUF,)),
          pltpu.SemaphoreType.DMA((NBUF,)),
          pltpu.SemaphoreType.DMA((NBUF,)),
          pltpu.VMEM_SHARED((N, d), jnp.float32),
      ],
      compiler_params=pltpu.CompilerParams(use_tc_tiling_on_sc=False),
  )
  def agg(table, src, dst, zeros, out, sbufs, dbufs, rbufs,
          isem, gsem, ssem, acc):
    cid = lax.axis_index("c")
    sid = lax.axis_index("s")
    wid = sid * NC + cid
    base = wid * EPW

    pltpu.sync_copy(zeros, acc.at[pl.ds(sid * RPW, RPW)])
    plsc.subcore_barrier()

    def fire_idx(c, b):
      pltpu.async_copy(src.at[pl.ds(base + c * chunk, chunk)], sbufs[b],
                       isem.at[b])
      pltpu.async_copy(dst.at[pl.ds(base + c * chunk, chunk)], dbufs[b],
                       isem.at[b])

    def fire_gather(c, b):
      pltpu.make_async_copy(src.at[pl.ds(base + c * chunk, chunk)], sbufs[b],
                            isem.at[b]).wait()
      pltpu.make_async_copy(dst.at[pl.ds(base + c * chunk, chunk)], dbufs[b],
                            isem.at[b]).wait()
      pltpu.async_copy(table.at[sbufs[b]], rbufs[b], gsem.at[b])

    for c in range(ILEAD):
      fire_idx(c, c)
    fire_gather(0, 0)

    def grp(g, carry):
      for k in range(NBUF):
        c = g * NBUF + k
        bs = (k + SLAG + 1) % NBUF
        bg = (k + 1) % NBUF
        @pl.when(c >= SLAG)
        def _():
          pltpu.make_async_copy(rbufs[bs], acc.at[dbufs[bs]],
                                ssem.at[bs]).wait()
        @pl.when(c + ILEAD < nchunk)
        def _():
          fire_idx(c + ILEAD, bs)
        @pl.when(c + 1 < nchunk)
        def _():
          fire_gather(c + 1, bg)
        pltpu.make_async_copy(table.at[sbufs[k]], rbufs[k], gsem.at[k]).wait()
        pltpu.async_copy(rbufs[k], acc.at[dbufs[k]], ssem.at[k], add=True)
      return carry

    lax.fori_loop(0, nchunk // NBUF, grp, 0)
    for c in range(nchunk - SLAG, nchunk):
      b = c % NBUF
      pltpu.make_async_copy(rbufs[b], acc.at[dbufs[b]], ssem.at[b]).wait()
    plsc.subcore_barrier()

    pltpu.sync_copy(acc.at[pl.ds(sid * RPW, RPW)],
                    out.at[cid, pl.ds(sid * RPW, RPW)])

  return agg


_sc_agg_cache = {}


def _agg(table, src, dst, zeros, d):
  if d not in _sc_agg_cache:
    _sc_agg_cache[d] = _make_sc_agg(d)
  return _sc_agg_cache[d](table, src, dst, zeros)


def _mm_body(x_ref, w_ref, o_ref):
  o_ref[...] = jnp.dot(x_ref[...], w_ref[...],
                       preferred_element_type=jnp.float32)


def _tc_matmul(x, w, d_out):
  n, d_in = x.shape
  return pl.pallas_call(
      _mm_body,
      grid=(n // ROW_BLK,),
      in_specs=[
          pl.BlockSpec((ROW_BLK, d_in), lambda i: (i, 0)),
          pl.BlockSpec((d_in, d_out), lambda i: (0, 0)),
      ],
      out_specs=pl.BlockSpec((ROW_BLK, d_out), lambda i: (i, 0)),
      out_shape=jax.ShapeDtypeStruct((n, d_out), jnp.float32),
  )(x, w)


def _stage2_body(p0_ref, p1_ref, b_ref, w_ref, t_ref, dinv_ref):
  s = p0_ref[...] + p1_ref[...]
  deg = s[:, D_H:D_H + 1]
  dinv = 1.0 / jnp.maximum(deg, 1.0)
  h = jnp.maximum(s[:, :D_H] * dinv + b_ref[...], 0.0)
  t_ref[...] = jnp.dot(h, w_ref[...], preferred_element_type=jnp.float32)
  dinv_ref[...] = dinv


def _tc_stage2(p0, p1, b0, w1):
  return pl.pallas_call(
      _stage2_body,
      grid=(GRID,),
      in_specs=[
          pl.BlockSpec((ROW_BLK, D0), lambda i: (i, 0)),
          pl.BlockSpec((ROW_BLK, D0), lambda i: (i, 0)),
          pl.BlockSpec((1, D_H), lambda i: (0, 0)),
          pl.BlockSpec((D_H, D_H), lambda i: (0, 0)),
      ],
      out_specs=[
          pl.BlockSpec((ROW_BLK, D_H), lambda i: (i, 0)),
          pl.BlockSpec((ROW_BLK, 1), lambda i: (i, 0)),
      ],
      out_shape=[
          jax.ShapeDtypeStruct((N, D_H), jnp.float32),
          jax.ShapeDtypeStruct((N, 1), jnp.float32),
      ],
  )(p0, p1, b0, w1)


def _stage3_body(p0_ref, p1_ref, dinv_ref, b_ref, w_ref, t_ref):
  s = p0_ref[...] + p1_ref[...]
  h = jnp.maximum(s * dinv_ref[...] + b_ref[...], 0.0)
  t_ref[...] = jnp.dot(h, w_ref[...], preferred_element_type=jnp.float32)


def _tc_stage3(p0, p1, dinv, b1, w2):
  return pl.pallas_call(
      _stage3_body,
      grid=(GRID,),
      in_specs=[
          pl.BlockSpec((ROW_BLK, D_H), lambda i: (i, 0)),
          pl.BlockSpec((ROW_BLK, D_H), lambda i: (i, 0)),
          pl.BlockSpec((ROW_BLK, 1), lambda i: (i, 0)),
          pl.BlockSpec((1, D_H), lambda i: (0, 0)),
          pl.BlockSpec((D_H, D2), lambda i: (0, 0)),
      ],
      out_specs=pl.BlockSpec((ROW_BLK, D2), lambda i: (i, 0)),
      out_shape=jax.ShapeDtypeStruct((N, D2), jnp.float32),
  )(p0, p1, dinv, b1, w2)


def _stage4_body(p0_ref, p1_ref, dinv_ref, b_ref, o_ref):
  s = p0_ref[...] + p1_ref[...]
  o_ref[...] = s * dinv_ref[...] + b_ref[...]


def _tc_stage4(p0, p1, dinv, b2):
  return pl.pallas_call(
      _stage4_body,
      grid=(GRID,),
      in_specs=[
          pl.BlockSpec((ROW_BLK, D2), lambda i: (i, 0)),
          pl.BlockSpec((ROW_BLK, D2), lambda i: (i, 0)),
          pl.BlockSpec((ROW_BLK, 1), lambda i: (i, 0)),
          pl.BlockSpec((1, D2), lambda i: (0, 0)),
      ],
      out_specs=pl.BlockSpec((ROW_BLK, D2), lambda i: (i, 0)),
      out_shape=jax.ShapeDtypeStruct((N, D2), jnp.float32),
  )(p0, p1, dinv, b2)


def kernel(features, edge_index, W0, b0, W1, b1, W2, b2):
  src = edge_index[0].astype(jnp.int32)
  dst = edge_index[1].astype(jnp.int32)

  xaug = jnp.concatenate(
      [features,
       jnp.ones((N, 1), jnp.float32),
       jnp.zeros((N, 7), jnp.float32)], axis=1)
  w0aug = jnp.zeros((D_IN + 8, D0), jnp.float32)
  w0aug = w0aug.at[:D_IN, :D_H].set(W0)
  w0aug = w0aug.at[D_IN, D_H].set(1.0)

  w2pad = jnp.zeros((D_H, D2), jnp.float32).at[:, :N_CLASSES].set(W2)
  b2pad = jnp.zeros((1, D2), jnp.float32).at[0, :N_CLASSES].set(b2)

  z0 = jnp.zeros((RPW, D0), jnp.float32)
  z1 = jnp.zeros((RPW, D1), jnp.float32)
  z2 = jnp.zeros((RPW, D2), jnp.float32)

  t0 = _tc_matmul(xaug, w0aug, D0)
  p = _agg(t0, src, dst, z0, D0)
  t1, dinv = _tc_stage2(p[0], p[1], b0.reshape(1, D_H), W1)
  p = _agg(t1, src, dst, z1, D1)
  t2 = _tc_stage3(p[0], p[1], dinv, b1.reshape(1, D_H), w2pad)
  p = _agg(t2, src, dst, z2, D2)
  out = _tc_stage4(p[0], p[1], dinv, b2pad)
  return out[:, :N_CLASSES]


# speedup vs baseline: 11.8434x; 1.1016x over previous
import functools

import jax
import jax.numpy as jnp
from jax import lax
from jax.experimental import pallas as pl
from jax.experimental.pallas import tpu as pltpu
from jax.experimental.pallas import tpu_sc as plsc

N = 10000
E = 320000
D_IN = 128
D_H = 128
N_CLASSES = 40

NC = 2
NS = 16
NW = NC * NS
EPW = E // NW
RPW = N // NS

D0 = 144
D1 = 128
D2 = 48

ROW_BLK = 2000
GRID = N // ROW_BLK


NBUF = 5
ILEAD = 3
GLEAD = 2
SLAG = 2
_CHUNK_BY_D = {144: 40, 128: 40, 48: 80}


def _make_sc_agg(d):
  chunk = _CHUNK_BY_D[d]
  nchunk = EPW // chunk
  mesh = plsc.VectorSubcoreMesh(core_axis_name="c", subcore_axis_name="s",
                                num_cores=NC, num_subcores=NS)

  @functools.partial(
      pl.kernel,
      out_type=jax.ShapeDtypeStruct((NC, N, d), jnp.float32),
      mesh=mesh,
      scratch_types=[
          [pltpu.VMEM((2, chunk), jnp.int32) for _ in range(NBUF)],
          [pltpu.VMEM((chunk, d), jnp.float32) for _ in range(NBUF)],
          pltpu.SemaphoreType.DMA((NBUF,)),
          pltpu.SemaphoreType.DMA((NBUF,)),
          pltpu.SemaphoreType.DMA((NBUF,)),
          pltpu.VMEM_SHARED((N, d), jnp.float32),
      ],
      compiler_params=pltpu.CompilerParams(use_tc_tiling_on_sc=False),
  )
  def agg(table, edges, zeros, out, ibufs, rbufs, isem, gsem, ssem, acc):
    cid = lax.axis_index("c")
    sid = lax.axis_index("s")
    wid = sid * NC + cid

    pltpu.sync_copy(zeros, acc.at[pl.ds(sid * RPW, RPW)])
    plsc.subcore_barrier()

    def fire_idx(c, b):
      pltpu.async_copy(edges.at[wid, c], ibufs[b], isem.at[b])

    def fire_gather(c, b):
      pltpu.make_async_copy(edges.at[wid, c], ibufs[b], isem.at[b]).wait()
      pltpu.async_copy(table.at[ibufs[b].at[0]], rbufs[b], gsem.at[b])

    for c in range(ILEAD):
      fire_idx(c, c)
    for c in range(GLEAD):
      fire_gather(c, c)

    def grp(g, carry):
      for k in range(NBUF):
        c = g * NBUF + k
        bs = (k + NBUF - SLAG) % NBUF
        bg = (k + GLEAD) % NBUF
        @pl.when(c >= SLAG)
        def _():
          pltpu.make_async_copy(rbufs[bs], acc.at[ibufs[bs].at[1]],
                                ssem.at[bs]).wait()
        @pl.when(c + ILEAD < nchunk)
        def _():
          fire_idx(c + ILEAD, (k + ILEAD) % NBUF)
        @pl.when(c + GLEAD < nchunk)
        def _():
          fire_gather(c + GLEAD, bg)
        pltpu.make_async_copy(table.at[ibufs[k].at[0]], rbufs[k],
                              gsem.at[k]).wait()
        pltpu.async_copy(rbufs[k], acc.at[ibufs[k].at[1]], ssem.at[k],
                         add=True)
      return carry

    lax.fori_loop(0, nchunk // NBUF, grp, 0)
    for c in range(nchunk - SLAG, nchunk):
      b = c % NBUF
      pltpu.make_async_copy(rbufs[b], acc.at[ibufs[b].at[1]],
                            ssem.at[b]).wait()
    plsc.subcore_barrier()

    pltpu.sync_copy(acc.at[pl.ds(sid * RPW, RPW)],
                    out.at[cid, pl.ds(sid * RPW, RPW)])

  return agg


_sc_agg_cache = {}


def _agg(table, edges_by_chunk, zeros, d):
  if d not in _sc_agg_cache:
    _sc_agg_cache[d] = _make_sc_agg(d)
  return _sc_agg_cache[d](table, edges_by_chunk[_CHUNK_BY_D[d]], zeros)


def _p_specs(d):
  return [
      pl.BlockSpec((1, ROW_BLK, d), lambda i: (0, i, 0)),
      pl.BlockSpec((1, ROW_BLK, d), lambda i: (1, i, 0)),
  ]


def _stage1_body(x_ref, w_ref, b_ref, o_ref):
  o_ref[...] = (jnp.dot(x_ref[...], w_ref[...],
                        preferred_element_type=jnp.float32) + b_ref[...])


def _tc_stage1(x, w, b):
  return pl.pallas_call(
      _stage1_body,
      grid=(GRID,),
      in_specs=[
          pl.BlockSpec((ROW_BLK, D_IN), lambda i: (i, 0)),
          pl.BlockSpec((D_IN, D0), lambda i: (0, 0)),
          pl.BlockSpec((1, D0), lambda i: (0, 0)),
      ],
      out_specs=pl.BlockSpec((ROW_BLK, D0), lambda i: (i, 0)),
      out_shape=jax.ShapeDtypeStruct((N, D0), jnp.float32),
  )(x, w, b)


def _stage2_body(p0_ref, p1_ref, b_ref, w_ref, t_ref, dinv_ref):
  s = p0_ref[0] + p1_ref[0]
  deg = s[:, D_H:D_H + 1]
  dinv = 1.0 / jnp.maximum(deg, 1.0)
  h = jnp.maximum(s[:, :D_H] * dinv + b_ref[...], 0.0)
  t_ref[...] = jnp.dot(h, w_ref[...], preferred_element_type=jnp.float32)
  dinv_ref[...] = dinv


def _tc_stage2(p, b0, w1):
  return pl.pallas_call(
      _stage2_body,
      grid=(GRID,),
      in_specs=_p_specs(D0) + [
          pl.BlockSpec((1, D_H), lambda i: (0, 0)),
          pl.BlockSpec((D_H, D_H), lambda i: (0, 0)),
      ],
      out_specs=[
          pl.BlockSpec((ROW_BLK, D_H), lambda i: (i, 0)),
          pl.BlockSpec((ROW_BLK, 1), lambda i: (i, 0)),
      ],
      out_shape=[
          jax.ShapeDtypeStruct((N, D_H), jnp.float32),
          jax.ShapeDtypeStruct((N, 1), jnp.float32),
      ],
  )(p, p, b0, w1)


def _stage3_body(p0_ref, p1_ref, dinv_ref, b_ref, w_ref, t_ref):
  s = p0_ref[0] + p1_ref[0]
  h = jnp.maximum(s * dinv_ref[...] + b_ref[...], 0.0)
  t_ref[...] = jnp.dot(h, w_ref[...], preferred_element_type=jnp.float32)


def _tc_stage3(p, dinv, b1, w2):
  return pl.pallas_call(
      _stage3_body,
      grid=(GRID,),
      in_specs=_p_specs(D1) + [
          pl.BlockSpec((ROW_BLK, 1), lambda i: (i, 0)),
          pl.BlockSpec((1, D_H), lambda i: (0, 0)),
          pl.BlockSpec((D_H, D2), lambda i: (0, 0)),
      ],
      out_specs=pl.BlockSpec((ROW_BLK, D2), lambda i: (i, 0)),
      out_shape=jax.ShapeDtypeStruct((N, D2), jnp.float32),
  )(p, p, dinv, b1, w2)


def _stage4_body(p0_ref, p1_ref, dinv_ref, b_ref, o_ref):
  s = p0_ref[0] + p1_ref[0]
  o_ref[...] = s * dinv_ref[...] + b_ref[...]


def _tc_stage4(p, dinv, b2):
  return pl.pallas_call(
      _stage4_body,
      grid=(GRID,),
      in_specs=_p_specs(D2) + [
          pl.BlockSpec((ROW_BLK, 1), lambda i: (i, 0)),
          pl.BlockSpec((1, D2), lambda i: (0, 0)),
      ],
      out_specs=pl.BlockSpec((ROW_BLK, D2), lambda i: (i, 0)),
      out_shape=jax.ShapeDtypeStruct((N, D2), jnp.float32),
  )(p, p, dinv, b2)


def kernel(features, edge_index, W0, b0, W1, b1, W2, b2):
  ei = edge_index.astype(jnp.int32)
  edges_by_chunk = {}
  for chunk in set(_CHUNK_BY_D.values()):
    nchunk = EPW // chunk
    edges_by_chunk[chunk] = (
        ei.reshape(2, NW, nchunk, chunk).transpose(1, 2, 0, 3))

  w0pad = jnp.zeros((D_IN, D0), jnp.float32).at[:, :D_H].set(W0)
  b0aug = jnp.zeros((1, D0), jnp.float32).at[0, D_H].set(1.0)

  w2pad = jnp.zeros((D_H, D2), jnp.float32).at[:, :N_CLASSES].set(W2)
  b2pad = jnp.zeros((1, D2), jnp.float32).at[0, :N_CLASSES].set(b2)

  z0 = jnp.zeros((RPW, D0), jnp.float32)
  z1 = jnp.zeros((RPW, D1), jnp.float32)
  z2 = jnp.zeros((RPW, D2), jnp.float32)

  t0 = _tc_stage1(features, w0pad, b0aug)
  p = _agg(t0, edges_by_chunk, z0, D0)
  t1, dinv = _tc_stage2(p, b0.reshape(1, D_H), W1)
  p = _agg(t1, edges_by_chunk, z1, D1)
  t2 = _tc_stage3(p, dinv, b1.reshape(1, D_H), w2pad)
  p = _agg(t2, edges_by_chunk, z2, D2)
  out = _tc_stage4(p, dinv, b2pad)
  return out[:, :N_CLASSES]


# speedup vs baseline: 13.2158x; 1.1159x over previous
import functools

import jax
import jax.numpy as jnp
from jax import lax
from jax.experimental import pallas as pl
from jax.experimental.pallas import tpu as pltpu
from jax.experimental.pallas import tpu_sc as plsc

N = 10000
E = 320000
D_IN = 128
D_H = 128
N_CLASSES = 40

NC = 2
NS = 16
NW = NC * NS
EPW = E // NW
RPW = N // NS

D0 = 144
D1 = 128
D2 = 48

ROW_BLK = 2000
GRID = N // ROW_BLK


NBUF = 5
ILEAD = 3
GLEAD = 2
SLAG = 2
_CHUNK_BY_D = {144: 40, 128: 40, 48: 80}


def _make_sc_agg(d):
  chunk = _CHUNK_BY_D[d]
  nchunk = EPW // chunk
  mesh = plsc.VectorSubcoreMesh(core_axis_name="c", subcore_axis_name="s",
                                num_cores=NC, num_subcores=NS)

  @functools.partial(
      pl.kernel,
      out_type=jax.ShapeDtypeStruct((NC, N, d), jnp.float32),
      mesh=mesh,
      scratch_types=[
          [pltpu.VMEM((2, chunk), jnp.int32) for _ in range(NBUF)],
          [pltpu.VMEM((chunk, d), jnp.float32) for _ in range(NBUF)],
          pltpu.SemaphoreType.DMA((NBUF,)),
          pltpu.SemaphoreType.DMA((NBUF,)),
          pltpu.SemaphoreType.DMA((NBUF,)),
          pltpu.VMEM_SHARED((N, d), jnp.float32),
      ],
      compiler_params=pltpu.CompilerParams(use_tc_tiling_on_sc=False),
  )
  def agg(table, edges, zeros, out, ibufs, rbufs, isem, gsem, ssem, acc):
    cid = lax.axis_index("c")
    sid = lax.axis_index("s")
    wid = sid * NC + cid

    pltpu.sync_copy(zeros, acc.at[pl.ds(sid * RPW, RPW)])
    plsc.subcore_barrier()

    base = wid * EPW

    def fire_idx(c, b):
      pltpu.async_copy(edges.at[:, pl.ds(base + c * chunk, chunk)], ibufs[b],
                       isem.at[b])

    def fire_gather(c, b):
      pltpu.make_async_copy(edges.at[:, pl.ds(base + c * chunk, chunk)],
                            ibufs[b], isem.at[b]).wait()
      pltpu.async_copy(table.at[ibufs[b].at[0]], rbufs[b], gsem.at[b])

    for c in range(ILEAD):
      fire_idx(c, c)
    for c in range(GLEAD):
      fire_gather(c, c)

    def grp(g, carry):
      for k in range(NBUF):
        c = g * NBUF + k
        bs = (k + NBUF - SLAG) % NBUF
        bg = (k + GLEAD) % NBUF
        @pl.when(c >= SLAG)
        def _():
          pltpu.make_async_copy(rbufs[bs], acc.at[ibufs[bs].at[1]],
                                ssem.at[bs]).wait()
        @pl.when(c + ILEAD < nchunk)
        def _():
          fire_idx(c + ILEAD, (k + ILEAD) % NBUF)
        @pl.when(c + GLEAD < nchunk)
        def _():
          fire_gather(c + GLEAD, bg)
        pltpu.make_async_copy(table.at[ibufs[k].at[0]], rbufs[k],
                              gsem.at[k]).wait()
        pltpu.async_copy(rbufs[k], acc.at[ibufs[k].at[1]], ssem.at[k],
                         add=True)
      return carry

    lax.fori_loop(0, nchunk // NBUF, grp, 0)
    for c in range(nchunk - SLAG, nchunk):
      b = c % NBUF
      pltpu.make_async_copy(rbufs[b], acc.at[ibufs[b].at[1]],
                            ssem.at[b]).wait()
    plsc.subcore_barrier()

    pltpu.sync_copy(acc.at[pl.ds(sid * RPW, RPW)],
                    out.at[cid, pl.ds(sid * RPW, RPW)])

  return agg


_sc_agg_cache = {}


def _agg(table, edges, zeros, d):
  if d not in _sc_agg_cache:
    _sc_agg_cache[d] = _make_sc_agg(d)
  return _sc_agg_cache[d](table, edges, zeros)


def _p_specs(d):
  return [
      pl.BlockSpec((1, ROW_BLK, d), lambda i: (0, i, 0)),
      pl.BlockSpec((1, ROW_BLK, d), lambda i: (1, i, 0)),
  ]


def _stage1_body(x_ref, w_ref, b_ref, o_ref):
  o_ref[...] = (jnp.dot(x_ref[...], w_ref[...],
                        preferred_element_type=jnp.float32) + b_ref[...])


def _tc_stage1(x, w, b):
  return pl.pallas_call(
      _stage1_body,
      grid=(GRID,),
      in_specs=[
          pl.BlockSpec((ROW_BLK, D_IN), lambda i: (i, 0)),
          pl.BlockSpec((D_IN, D0), lambda i: (0, 0)),
          pl.BlockSpec((1, D0), lambda i: (0, 0)),
      ],
      out_specs=pl.BlockSpec((ROW_BLK, D0), lambda i: (i, 0)),
      out_shape=jax.ShapeDtypeStruct((N, D0), jnp.float32),
  )(x, w, b)


def _stage2_body(p0_ref, p1_ref, b_ref, w_ref, t_ref, dinv_ref):
  s = p0_ref[0] + p1_ref[0]
  deg = s[:, D_H:D_H + 1]
  dinv = 1.0 / jnp.maximum(deg, 1.0)
  h = jnp.maximum(s[:, :D_H] * dinv + b_ref[...], 0.0)
  t_ref[...] = jnp.dot(h, w_ref[...], preferred_element_type=jnp.float32)
  dinv_ref[...] = dinv


def _tc_stage2(p, b0, w1):
  return pl.pallas_call(
      _stage2_body,
      grid=(GRID,),
      in_specs=_p_specs(D0) + [
          pl.BlockSpec((1, D_H), lambda i: (0, 0)),
          pl.BlockSpec((D_H, D_H), lambda i: (0, 0)),
      ],
      out_specs=[
          pl.BlockSpec((ROW_BLK, D_H), lambda i: (i, 0)),
          pl.BlockSpec((ROW_BLK, 1), lambda i: (i, 0)),
      ],
      out_shape=[
          jax.ShapeDtypeStruct((N, D_H), jnp.float32),
          jax.ShapeDtypeStruct((N, 1), jnp.float32),
      ],
  )(p, p, b0, w1)


def _stage3_body(p0_ref, p1_ref, dinv_ref, b_ref, w_ref, t_ref):
  s = p0_ref[0] + p1_ref[0]
  h = jnp.maximum(s * dinv_ref[...] + b_ref[...], 0.0)
  t_ref[...] = jnp.dot(h, w_ref[...], preferred_element_type=jnp.float32)


def _tc_stage3(p, dinv, b1, w2):
  return pl.pallas_call(
      _stage3_body,
      grid=(GRID,),
      in_specs=_p_specs(D1) + [
          pl.BlockSpec((ROW_BLK, 1), lambda i: (i, 0)),
          pl.BlockSpec((1, D_H), lambda i: (0, 0)),
          pl.BlockSpec((D_H, D2), lambda i: (0, 0)),
      ],
      out_specs=pl.BlockSpec((ROW_BLK, D2), lambda i: (i, 0)),
      out_shape=jax.ShapeDtypeStruct((N, D2), jnp.float32),
  )(p, p, dinv, b1, w2)


def _stage4_body(p0_ref, p1_ref, dinv_ref, b_ref, o_ref):
  s = p0_ref[0, :, :N_CLASSES] + p1_ref[0, :, :N_CLASSES]
  o_ref[...] = s * dinv_ref[...] + b_ref[...]


def _tc_stage4(p, dinv, b2):
  return pl.pallas_call(
      _stage4_body,
      grid=(GRID,),
      in_specs=_p_specs(D2) + [
          pl.BlockSpec((ROW_BLK, 1), lambda i: (i, 0)),
          pl.BlockSpec((1, N_CLASSES), lambda i: (0, 0)),
      ],
      out_specs=pl.BlockSpec((ROW_BLK, N_CLASSES), lambda i: (i, 0)),
      out_shape=jax.ShapeDtypeStruct((N, N_CLASSES), jnp.float32),
  )(p, p, dinv, b2)


def kernel(features, edge_index, W0, b0, W1, b1, W2, b2):
  ei = edge_index.astype(jnp.int32)

  w0pad = jnp.zeros((D_IN, D0), jnp.float32).at[:, :D_H].set(W0)
  b0aug = jnp.zeros((1, D0), jnp.float32).at[0, D_H].set(1.0)

  w2pad = jnp.zeros((D_H, D2), jnp.float32).at[:, :N_CLASSES].set(W2)

  z0 = jnp.zeros((RPW, D0), jnp.float32)
  z1 = jnp.zeros((RPW, D1), jnp.float32)
  z2 = jnp.zeros((RPW, D2), jnp.float32)

  t0 = _tc_stage1(features, w0pad, b0aug)
  p = _agg(t0, ei, z0, D0)
  t1, dinv = _tc_stage2(p, b0.reshape(1, D_H), W1)
  p = _agg(t1, ei, z1, D1)
  t2 = _tc_stage3(p, dinv, b1.reshape(1, D_H), w2pad)
  p = _agg(t2, ei, z2, D2)
  return _tc_stage4(p, dinv, b2.reshape(1, N_CLASSES))
